# baseline (speedup 1.0000x reference)
import functools

import jax
import jax.numpy as jnp
import numpy as np
from jax.experimental import pallas as pl

WS = 8
B, H, W = 16, 512, 512
GH, GW = H // WS, W // WS
NCELL = GH * GW
CELL = WS * WS
PER_BATCH_CAT = NCELL * CELL

K1 = (np.uint32(928981903), np.uint32(3453687069))
K2 = (np.uint32(4146024105), np.uint32(2718843009))
TINY = np.float32(np.finfo(np.float32).tiny)


def _rotl(x, d):
    return (x << np.uint32(d)) | (x >> np.uint32(32 - d))


def _threefry_bits(key, x1):
    k0, k1 = key
    ks = (k0, k1, np.uint32(np.uint32(k0) ^ np.uint32(k1) ^ np.uint32(0x1BD11BDA)))
    rot = ((13, 15, 26, 6), (17, 29, 16, 24))
    x0 = jnp.full_like(x1, ks[0])
    x1 = x1 + ks[1]
    for i in range(5):
        for r in rot[i % 2]:
            x0 = x0 + x1
            x1 = _rotl(x1, r) ^ x0
        x0 = x0 + ks[(i + 1) % 3]
        x1 = x1 + ks[(i + 2) % 3] + np.uint32(i + 1)
    return x0 ^ x1


def _u01(bits):
    f = jax.lax.bitcast_convert_type(
        (bits >> np.uint32(9)) | np.uint32(0x3F800000), jnp.float32)
    return f - jnp.float32(1.0)


def _log_sigmoid(x):
    return jnp.minimum(x, 0.0) - jnp.log1p(jnp.exp(-jnp.abs(x)))


CH, CW = 8, 512


def _body(base, ct_ref, lp_ref, acc_ref, xf_ref, yf_ref):
    b = pl.program_id(0).astype(jnp.uint32) + np.uint32(base)
    a = ct_ref[0]

    shp = (CELL, CH, CW)
    ci = jax.lax.broadcasted_iota(jnp.uint32, shp, 0)
    cell = (jax.lax.broadcasted_iota(jnp.uint32, shp, 1) * np.uint32(CW)
            + jax.lax.broadcasted_iota(jnp.uint32, shp, 2))
    n = cell * np.uint32(CELL) + ci + b * np.uint32(PER_BATCH_CAT)
    u = _u01(_threefry_bits(K1, n)) + TINY
    score = a - jnp.log(-jnp.log(u))

    mx = jnp.max(score, axis=0)
    lanes = ci.astype(jnp.int32)
    choice = jnp.min(jnp.where(score == mx[None], lanes, CELL), axis=0)
    chm = lanes == choice[None]

    selected = jnp.sum(jnp.where(chm, a, 0.0), axis=0)
    xmax = jnp.max(a, axis=0)
    sumexp = jnp.sum(jnp.exp(a - xmax[None]), axis=0)
    logp_cat = (selected - xmax) - jnp.log(sumexp)

    shp2 = (CH, CW)
    cell2 = (jax.lax.broadcasted_iota(jnp.uint32, shp2, 0) * np.uint32(CW)
             + jax.lax.broadcasted_iota(jnp.uint32, shp2, 1))
    u2 = _u01(_threefry_bits(K2, cell2 + b * np.uint32(NCELL)))
    p = jax.nn.sigmoid(selected)
    acc = (u2 < p).astype(jnp.float32)

    logp_bern = acc * _log_sigmoid(selected) + (1.0 - acc) * _log_sigmoid(-selected)
    lp_ref[0] = logp_cat + logp_bern
    acc_ref[0] = acc

    celli = cell2.astype(jnp.int32)
    xf_ref[0] = ((celli & 63) * WS + (choice & 7)).astype(jnp.float32)
    yf_ref[0] = ((celli >> 6) * WS + (choice >> 3)).astype(jnp.float32)


CHUNK = 8


def _run(x, interpret=False):
    xr = x.reshape(B, GH, WS, GW, WS)
    out = jax.ShapeDtypeStruct((CHUNK, CH, CW), jnp.float32)
    ospec = pl.BlockSpec((1, CH, CW), lambda b: (b, 0, 0))
    parts = []
    for s in range(0, B, CHUNK):
        ct = jnp.transpose(
            xr[s:s + CHUNK], (0, 2, 4, 1, 3)).reshape(CHUNK, CELL, CH, CW)
        parts.append(pl.pallas_call(
            functools.partial(_body, s),
            grid=(CHUNK,),
            in_specs=[pl.BlockSpec((1, CELL, CH, CW), lambda b: (b, 0, 0, 0))],
            out_specs=[ospec, ospec, ospec, ospec],
            out_shape=[out, out, out, out],
            interpret=interpret,
        )(ct))
    lp, acc, xf, yf = (jnp.concatenate(t, axis=0) for t in zip(*parts))
    lp = lp.reshape(B, GH, GW)
    acc = acc.reshape(B, GH, GW)
    xy = jnp.stack([xf.reshape(B, GH, GW), yf.reshape(B, GH, GW)], axis=-1)
    return xy, lp, acc > 0


def kernel(x):
    return _run(x)


# speedup vs baseline: 1.7533x; 1.7533x over previous
import jax
import jax.numpy as jnp
import numpy as np
from jax.experimental import pallas as pl

WS = 8
B, H, W = 16, 512, 512
GH, GW = H // WS, W // WS
NCELL = GH * GW
CELL = WS * WS
PER_BATCH_CAT = NCELL * CELL

K1 = (np.uint32(928981903), np.uint32(3453687069))
K2 = (np.uint32(4146024105), np.uint32(2718843009))
TINY = np.float32(np.finfo(np.float32).tiny)


def _rotl(x, d):
    return (x << np.uint32(d)) | (x >> np.uint32(32 - d))


def _threefry_bits(key, x1):
    k0, k1 = key
    ks = (k0, k1, np.uint32(np.uint32(k0) ^ np.uint32(k1) ^ np.uint32(0x1BD11BDA)))
    rot = ((13, 15, 26, 6), (17, 29, 16, 24))
    x0 = jnp.full_like(x1, ks[0])
    x1 = x1 + ks[1]
    for i in range(5):
        for r in rot[i % 2]:
            x0 = x0 + x1
            x1 = _rotl(x1, r) ^ x0
        x0 = x0 + ks[(i + 1) % 3]
        x1 = x1 + ks[(i + 2) % 3] + np.uint32(i + 1)
    return x0 ^ x1


def _u01(bits):
    f = jax.lax.bitcast_convert_type(
        (bits >> np.uint32(9)) | np.uint32(0x3F800000), jnp.float32)
    return f - jnp.float32(1.0)


def _log_sigmoid(x):
    return jnp.minimum(x, 0.0) - jnp.log1p(jnp.exp(-jnp.abs(x)))


CH, CW = 8, 512


def _body(ct_ref, lp_ref, acc_ref, xf_ref, yf_ref):
    b = pl.program_id(0).astype(jnp.uint32)
    a = ct_ref[0]

    shp = (CELL, CH, CW)
    ci = jax.lax.broadcasted_iota(jnp.uint32, shp, 0)
    cell = (jax.lax.broadcasted_iota(jnp.uint32, shp, 1) * np.uint32(CW)
            + jax.lax.broadcasted_iota(jnp.uint32, shp, 2))
    n = cell * np.uint32(CELL) + ci + b * np.uint32(PER_BATCH_CAT)
    u = _u01(_threefry_bits(K1, n)) + TINY
    score = a - jnp.log(-jnp.log(u))

    mx = jnp.max(score, axis=0)
    lanes = ci.astype(jnp.int32)
    choice = jnp.min(jnp.where(score == mx[None], lanes, CELL), axis=0)
    chm = lanes == choice[None]

    selected = jnp.sum(jnp.where(chm, a, 0.0), axis=0)
    xmax = jnp.max(a, axis=0)
    sumexp = jnp.sum(jnp.exp(a - xmax[None]), axis=0)
    logp_cat = (selected - xmax) - jnp.log(sumexp)

    shp2 = (CH, CW)
    cell2 = (jax.lax.broadcasted_iota(jnp.uint32, shp2, 0) * np.uint32(CW)
             + jax.lax.broadcasted_iota(jnp.uint32, shp2, 1))
    u2 = _u01(_threefry_bits(K2, cell2 + b * np.uint32(NCELL)))
    p = jax.nn.sigmoid(selected)
    acc = (u2 < p).astype(jnp.float32)

    logp_bern = acc * _log_sigmoid(selected) + (1.0 - acc) * _log_sigmoid(-selected)
    lp_ref[0] = logp_cat + logp_bern
    acc_ref[0] = acc

    celli = cell2.astype(jnp.int32)
    xf_ref[0] = ((celli & 63) * WS + (choice & 7)).astype(jnp.float32)
    yf_ref[0] = ((celli >> 6) * WS + (choice >> 3)).astype(jnp.float32)


def _run(x, interpret=False):
    ct = jnp.transpose(
        x.reshape(B, GH, WS, GW, WS), (0, 2, 4, 1, 3)).reshape(B, CELL, CH, CW)
    out = jax.ShapeDtypeStruct((B, CH, CW), jnp.float32)
    ospec = pl.BlockSpec((1, CH, CW), lambda b: (b, 0, 0))
    lp, acc, xf, yf = pl.pallas_call(
        _body,
        grid=(B,),
        in_specs=[pl.BlockSpec((1, CELL, CH, CW), lambda b: (b, 0, 0, 0))],
        out_specs=[ospec, ospec, ospec, ospec],
        out_shape=[out, out, out, out],
        interpret=interpret,
    )(ct)
    lp = lp.reshape(B, GH, GW)
    acc = acc.reshape(B, GH, GW)
    xy = jnp.stack([xf.reshape(B, GH, GW), yf.reshape(B, GH, GW)], axis=-1)
    return xy, lp, acc > 0


def kernel(x):
    return _run(x)


# speedup vs baseline: 1.7540x; 1.0004x over previous
import jax
import jax.numpy as jnp
import numpy as np
from jax.experimental import pallas as pl

WS = 8
B, H, W = 16, 512, 512
GH, GW = H // WS, W // WS
NCELL = GH * GW
CELL = WS * WS
PER_BATCH_CAT = NCELL * CELL

K1 = (np.uint32(928981903), np.uint32(3453687069))
K2 = (np.uint32(4146024105), np.uint32(2718843009))
TINY = np.float32(np.finfo(np.float32).tiny)


def _rotl(x, d):
    return (x << np.uint32(d)) | (x >> np.uint32(32 - d))


def _threefry_bits(key, x1):
    k0, k1 = key
    ks = (k0, k1, np.uint32(np.uint32(k0) ^ np.uint32(k1) ^ np.uint32(0x1BD11BDA)))
    rot = ((13, 15, 26, 6), (17, 29, 16, 24))
    x0 = jnp.full_like(x1, ks[0])
    x1 = x1 + ks[1]
    for i in range(5):
        for r in rot[i % 2]:
            x0 = x0 + x1
            x1 = _rotl(x1, r) ^ x0
        x0 = x0 + ks[(i + 1) % 3]
        x1 = x1 + ks[(i + 2) % 3] + np.uint32(i + 1)
    return x0 ^ x1


def _u01(bits):
    f = jax.lax.bitcast_convert_type(
        (bits >> np.uint32(9)) | np.uint32(0x3F800000), jnp.float32)
    return f - jnp.float32(1.0)


def _log_sigmoid(x):
    return jnp.minimum(x, 0.0) - jnp.log1p(jnp.exp(-jnp.abs(x)))


CH, CW = 8, 512


def _body(ct_ref, lp_ref, acc_ref, xf_ref, yf_ref):
    b = pl.program_id(0).astype(jnp.uint32)
    a = ct_ref[0]

    shp = (CELL, CH, CW)
    ci = jax.lax.broadcasted_iota(jnp.uint32, shp, 0)
    cell = (jax.lax.broadcasted_iota(jnp.uint32, shp, 1) * np.uint32(CW)
            + jax.lax.broadcasted_iota(jnp.uint32, shp, 2))
    n = cell * np.uint32(CELL) + ci + b * np.uint32(PER_BATCH_CAT)
    u = _u01(_threefry_bits(K1, n)) + TINY
    score = a - jnp.log(-jnp.log(u))

    mx = jnp.max(score, axis=0)
    lanes = ci.astype(jnp.int32)
    choice = jnp.min(jnp.where(score == mx[None], lanes, CELL), axis=0)
    chm = lanes == choice[None]

    selected = jnp.sum(jnp.where(chm, a, 0.0), axis=0)
    xmax = jnp.max(a, axis=0)
    sumexp = jnp.sum(jnp.exp(a - xmax[None]), axis=0)
    logp_cat = (selected - xmax) - jnp.log(sumexp)

    shp2 = (CH, CW)
    cell2 = (jax.lax.broadcasted_iota(jnp.uint32, shp2, 0) * np.uint32(CW)
             + jax.lax.broadcasted_iota(jnp.uint32, shp2, 1))
    u2 = _u01(_threefry_bits(K2, cell2 + b * np.uint32(NCELL)))
    p = jax.nn.sigmoid(selected)
    acc = (u2 < p).astype(jnp.float32)

    logp_bern = acc * _log_sigmoid(selected) + (1.0 - acc) * _log_sigmoid(-selected)
    lp_ref[0] = logp_cat + logp_bern
    acc_ref[0] = acc

    celli = cell2.astype(jnp.int32)
    xf_ref[0] = ((celli & 63) * WS + (choice & 7)).astype(jnp.float32)
    yf_ref[0] = ((celli >> 6) * WS + (choice >> 3)).astype(jnp.float32)


def kernel(x):
    ct = jnp.transpose(
        x.reshape(B, GH, WS, GW, WS), (0, 2, 4, 1, 3)).reshape(B, CELL, CH, CW)
    out = jax.ShapeDtypeStruct((B, CH, CW), jnp.float32)
    ospec = pl.BlockSpec((1, CH, CW), lambda b: (b, 0, 0))
    lp, acc, xf, yf = pl.pallas_call(
        _body,
        grid=(B,),
        in_specs=[pl.BlockSpec((1, CELL, CH, CW), lambda b: (b, 0, 0, 0))],
        out_specs=[ospec, ospec, ospec, ospec],
        out_shape=[out, out, out, out],
    )(ct)
    lp = lp.reshape(B, GH, GW)
    acc = acc.reshape(B, GH, GW)
    xy = jnp.stack([xf.reshape(B, GH, GW), yf.reshape(B, GH, GW)], axis=-1)
    return xy, lp, acc > 0
